# baseline (speedup 1.0000x reference)
import functools

import jax
import jax.numpy as jnp
from jax import lax
from jax.experimental import pallas as pl
from jax.experimental.pallas import tpu as pltpu
from jax.experimental.pallas import tpu_sc as plsc

D = 768
DH = D // 2
S = 512
B = 128
N = S * B
EPS = 1e-12

NC = 2
NS = 16
NW = NC * NS
CHUNK = 128
NBUF = 2

K = 2
SC_CHUNK = S // K
NROWS = N // K
PER_W = NROWS // NW
NCHUNK = PER_W // CHUNK


def _sc_gather(table, ids_1d, width, dtype):
    mesh = plsc.VectorSubcoreMesh(core_axis_name="core", subcore_axis_name="subcore")

    @functools.partial(
        pl.kernel,
        out_type=jax.ShapeDtypeStruct((NROWS, width), dtype),
        mesh=mesh,
        scratch_types=[
            pltpu.VMEM((PER_W,), jnp.int32),
            pltpu.VMEM((NBUF, CHUNK, width), dtype),
            pltpu.SemaphoreType.DMA,
            pltpu.SemaphoreType.DMA,
            pltpu.SemaphoreType.DMA,
            pltpu.SemaphoreType.DMA,
        ],
    )
    def gather_kernel(x_hbm, i_hbm, o_hbm, idx_v, rows_v, g0, g1, s0, s1):
        gsem = (g0, g1)
        ssem = (s0, s1)
        wid = lax.axis_index("subcore") * NC + lax.axis_index("core")
        base = pl.multiple_of(wid * PER_W, PER_W)
        pltpu.sync_copy(i_hbm.at[pl.ds(base, PER_W)], idx_v)

        def fire_gather(c, b):
            return pltpu.async_copy(
                x_hbm.at[idx_v.at[pl.ds(c * CHUNK, CHUNK)]], rows_v.at[b], gsem[b]
            )

        def fire_store(c, b):
            return pltpu.async_copy(
                rows_v.at[b], o_hbm.at[pl.ds(base + c * CHUNK, CHUNK)], ssem[b]
            )

        def wait_store(b):
            pltpu.make_async_copy(
                rows_v.at[b], o_hbm.at[pl.ds(0, CHUNK)], ssem[b]
            ).wait()

        @pl.loop(0, NCHUNK, step=NBUF)
        def _(c0):
            @pl.when(c0 != 0)
            def _():
                wait_store(0)

            g_a = fire_gather(c0, 0)

            @pl.when(c0 != 0)
            def _():
                wait_store(1)

            g_b = fire_gather(c0 + 1, 1)
            g_a.wait()
            fire_store(c0, 0)
            g_b.wait()
            fire_store(c0 + 1, 1)

        wait_store(0)
        wait_store(1)

    return gather_kernel(table, ids_1d)


RB = 32


def _ln_body(x_ref, pos_ref, tok_ref, g_ref, b_ref, *rest):
    o_ref = rest[-1]
    x = x_ref[...]
    lo = lax.bitcast_convert_type(x << 16, jnp.float32)
    hi = lax.bitcast_convert_type(x, jnp.float32)
    bias = pos_ref[...] + tok_ref[...]
    emb_lo = lo + bias[:, None, :DH]
    emb_hi = hi + bias[:, None, DH:]
    tot = jnp.sum(emb_lo, axis=-1, keepdims=True) + jnp.sum(
        emb_hi, axis=-1, keepdims=True
    )
    mean = tot / D
    c_lo = emb_lo - mean
    c_hi = emb_hi - mean
    var = (
        jnp.sum(c_lo * c_lo, axis=-1, keepdims=True)
        + jnp.sum(c_hi * c_hi, axis=-1, keepdims=True)
    ) / D
    rstd = lax.rsqrt(var + EPS)
    g = g_ref[...][0]
    b = b_ref[...][0]
    o_ref[:, :, :DH] = c_lo * rstd * g[None, None, :DH] + b[None, None, :DH]
    o_ref[:, :, DH:] = c_hi * rstd * g[None, None, DH:] + b[None, None, DH:]


def _ln_body_f32(x_ref, pos_ref, tok_ref, g_ref, b_ref, *rest):
    o_ref = rest[-1]
    x = x_ref[...]
    bias = pos_ref[...] + tok_ref[...]
    emb = x + bias[:, None, :]
    mean = jnp.mean(emb, axis=-1, keepdims=True)
    c = emb - mean
    var = jnp.mean(c * c, axis=-1, keepdims=True)
    rstd = lax.rsqrt(var + EPS)
    g = g_ref[...][0]
    b = b_ref[...][0]
    o_ref[...] = c * rstd * g[None, None, :] + b[None, None, :]


NBLK = SC_CHUNK // RB


def _tc_ln_chunk(g3c, pos_c, tok_row, gamma2, beta2, buf, k, packed_in=True):
    body = _ln_body if packed_in else _ln_body_f32
    xw = DH if packed_in else D
    rb = RB if packed_in else RB // 2
    nblk = SC_CHUNK // rb
    in_specs = [
        pl.BlockSpec((rb, B, xw), lambda i: (i, 0, 0)),
        pl.BlockSpec((rb, D), lambda i: (i, 0)),
        pl.BlockSpec((1, D), lambda i: (0, 0)),
        pl.BlockSpec((1, D), lambda i: (0, 0)),
        pl.BlockSpec((1, D), lambda i: (0, 0)),
    ]
    args = [g3c, pos_c, tok_row, gamma2, beta2]
    aliases = {}
    if buf is not None:
        in_specs.append(pl.BlockSpec(memory_space=pl.ANY))
        args.append(buf)
        aliases = {5: 0}
    return pl.pallas_call(
        body,
        grid=(nblk,),
        in_specs=in_specs,
        out_specs=pl.BlockSpec(
            (rb, B, D), lambda i, k=k, nblk=nblk: (i + k * nblk, 0, 0)
        ),
        out_shape=jax.ShapeDtypeStruct((S, B, D), jnp.float32),
        input_output_aliases=aliases,
        compiler_params=pltpu.CompilerParams(
            dimension_semantics=("arbitrary",),
        ),
    )(*args)


def _pack_table(word_emb):
    def rne16(half):
        u = lax.bitcast_convert_type(half, jnp.uint32)
        return (u + jnp.uint32(0x7FFF) + ((u >> 16) & jnp.uint32(1))) >> 16

    lo = rne16(word_emb[:, :DH])
    hi = rne16(word_emb[:, DH:])
    return lax.bitcast_convert_type(lo | (hi << 16), jnp.int32)


def kernel(input_ids, word_emb, pos_emb, tok_emb, ln_gamma, ln_beta):
    ids1 = input_ids.astype(jnp.int32).reshape(N)
    packed = _pack_table(word_emb)
    tok_row = tok_emb[0:1]
    gamma2 = ln_gamma.reshape(1, D)
    beta2 = ln_beta.reshape(1, D)

    gathered = [
        _sc_gather(packed, ids1[k * NROWS:(k + 1) * NROWS], DH, jnp.int32)
        for k in range(K)
    ]
    buf = None
    for k in range(K):
        g3c = gathered[k].reshape(SC_CHUNK, B, DH)
        pos_c = pos_emb[k * SC_CHUNK:(k + 1) * SC_CHUNK]
        buf = _tc_ln_chunk(g3c, pos_c, tok_row, gamma2, beta2, buf, k)
    return buf


# speedup vs baseline: 1.0046x; 1.0046x over previous
import functools

import jax
import jax.numpy as jnp
from jax import lax
from jax.experimental import pallas as pl
from jax.experimental.pallas import tpu as pltpu
from jax.experimental.pallas import tpu_sc as plsc

D = 768
DH = D // 2
S = 512
B = 128
N = S * B
EPS = 1e-12

NC = 2
NS = 16
NW = NC * NS
CHUNK = 128
NBUF = 2

K = 4
SC_CHUNK = S // K
NROWS = N // K
PER_W = NROWS // NW
NCHUNK = PER_W // CHUNK


def _sc_gather(table, ids_1d, width, dtype):
    mesh = plsc.VectorSubcoreMesh(core_axis_name="core", subcore_axis_name="subcore")

    @functools.partial(
        pl.kernel,
        out_type=jax.ShapeDtypeStruct((NROWS, width), dtype),
        mesh=mesh,
        scratch_types=[
            pltpu.VMEM((PER_W,), jnp.int32),
            pltpu.VMEM((NBUF, CHUNK, width), dtype),
            pltpu.SemaphoreType.DMA,
            pltpu.SemaphoreType.DMA,
            pltpu.SemaphoreType.DMA,
            pltpu.SemaphoreType.DMA,
        ],
    )
    def gather_kernel(x_hbm, i_hbm, o_hbm, idx_v, rows_v, g0, g1, s0, s1):
        gsem = (g0, g1)
        ssem = (s0, s1)
        wid = lax.axis_index("subcore") * NC + lax.axis_index("core")
        base = pl.multiple_of(wid * PER_W, PER_W)
        pltpu.sync_copy(i_hbm.at[pl.ds(base, PER_W)], idx_v)

        def fire_gather(c, b):
            return pltpu.async_copy(
                x_hbm.at[idx_v.at[pl.ds(c * CHUNK, CHUNK)]], rows_v.at[b], gsem[b]
            )

        def fire_store(c, b):
            return pltpu.async_copy(
                rows_v.at[b], o_hbm.at[pl.ds(base + c * CHUNK, CHUNK)], ssem[b]
            )

        def wait_store(b):
            pltpu.make_async_copy(
                rows_v.at[b], o_hbm.at[pl.ds(0, CHUNK)], ssem[b]
            ).wait()

        @pl.loop(0, NCHUNK, step=NBUF)
        def _(c0):
            @pl.when(c0 != 0)
            def _():
                wait_store(0)

            g_a = fire_gather(c0, 0)

            @pl.when(c0 != 0)
            def _():
                wait_store(1)

            g_b = fire_gather(c0 + 1, 1)
            g_a.wait()
            fire_store(c0, 0)
            g_b.wait()
            fire_store(c0 + 1, 1)

        wait_store(0)
        wait_store(1)

    return gather_kernel(table, ids_1d)


RB = 32


def _ln_body(x_ref, pos_ref, tok_ref, g_ref, b_ref, *rest):
    o_ref = rest[-1]
    x = x_ref[...]
    lo = lax.bitcast_convert_type(x << 16, jnp.float32)
    hi = lax.bitcast_convert_type(x, jnp.float32)
    bias = pos_ref[...] + tok_ref[...]
    emb_lo = lo + bias[:, None, :DH]
    emb_hi = hi + bias[:, None, DH:]
    tot = jnp.sum(emb_lo, axis=-1, keepdims=True) + jnp.sum(
        emb_hi, axis=-1, keepdims=True
    )
    mean = tot / D
    c_lo = emb_lo - mean
    c_hi = emb_hi - mean
    var = (
        jnp.sum(c_lo * c_lo, axis=-1, keepdims=True)
        + jnp.sum(c_hi * c_hi, axis=-1, keepdims=True)
    ) / D
    rstd = lax.rsqrt(var + EPS)
    g = g_ref[...][0]
    b = b_ref[...][0]
    o_ref[:, :, :DH] = c_lo * rstd * g[None, None, :DH] + b[None, None, :DH]
    o_ref[:, :, DH:] = c_hi * rstd * g[None, None, DH:] + b[None, None, DH:]


def _ln_body_f32(x_ref, pos_ref, tok_ref, g_ref, b_ref, *rest):
    o_ref = rest[-1]
    x = x_ref[...]
    bias = pos_ref[...] + tok_ref[...]
    emb = x + bias[:, None, :]
    mean = jnp.mean(emb, axis=-1, keepdims=True)
    c = emb - mean
    var = jnp.mean(c * c, axis=-1, keepdims=True)
    rstd = lax.rsqrt(var + EPS)
    g = g_ref[...][0]
    b = b_ref[...][0]
    o_ref[...] = c * rstd * g[None, None, :] + b[None, None, :]


NBLK = SC_CHUNK // RB


def _tc_ln_chunk(g3c, pos_c, tok_row, gamma2, beta2, buf, k, packed_in=True):
    body = _ln_body if packed_in else _ln_body_f32
    xw = DH if packed_in else D
    rb = RB if packed_in else RB // 2
    nblk = SC_CHUNK // rb
    in_specs = [
        pl.BlockSpec((rb, B, xw), lambda i: (i, 0, 0)),
        pl.BlockSpec((rb, D), lambda i: (i, 0)),
        pl.BlockSpec((1, D), lambda i: (0, 0)),
        pl.BlockSpec((1, D), lambda i: (0, 0)),
        pl.BlockSpec((1, D), lambda i: (0, 0)),
    ]
    args = [g3c, pos_c, tok_row, gamma2, beta2]
    aliases = {}
    if buf is not None:
        in_specs.append(pl.BlockSpec(memory_space=pl.ANY))
        args.append(buf)
        aliases = {5: 0}
    return pl.pallas_call(
        body,
        grid=(nblk,),
        in_specs=in_specs,
        out_specs=pl.BlockSpec(
            (rb, B, D), lambda i, k=k, nblk=nblk: (i + k * nblk, 0, 0)
        ),
        out_shape=jax.ShapeDtypeStruct((S, B, D), jnp.float32),
        input_output_aliases=aliases,
        compiler_params=pltpu.CompilerParams(
            dimension_semantics=("arbitrary",),
        ),
    )(*args)


def _pack_table(word_emb):
    def rne16(half):
        u = lax.bitcast_convert_type(half, jnp.uint32)
        return (u + jnp.uint32(0x7FFF) + ((u >> 16) & jnp.uint32(1))) >> 16

    lo = rne16(word_emb[:, :DH])
    hi = rne16(word_emb[:, DH:])
    return lax.bitcast_convert_type(lo | (hi << 16), jnp.int32)


def kernel(input_ids, word_emb, pos_emb, tok_emb, ln_gamma, ln_beta):
    ids1 = input_ids.astype(jnp.int32).reshape(N)
    packed = _pack_table(word_emb)
    tok_row = tok_emb[0:1]
    gamma2 = ln_gamma.reshape(1, D)
    beta2 = ln_beta.reshape(1, D)

    gathered = [
        _sc_gather(packed, ids1[k * NROWS:(k + 1) * NROWS], DH, jnp.int32)
        for k in range(K)
    ]
    buf = None
    for k in range(K):
        g3c = gathered[k].reshape(SC_CHUNK, B, DH)
        pos_c = pos_emb[k * SC_CHUNK:(k + 1) * SC_CHUNK]
        buf = _tc_ln_chunk(g3c, pos_c, tok_row, gamma2, beta2, buf, k)
    return buf
